# baseline (speedup 1.0000x reference)
import jax
import jax.numpy as jnp
from jax import lax
from jax.experimental import pallas as pl
from jax.experimental.pallas import tpu as pltpu
from jax.experimental.pallas import tpu_sc as plsc

_B = 4096
_D = 64
_K = 8192
_COMMITMENT_COST = 0.25

_NCHUNK = 2
_CB = _B // _NCHUNK

_BB = 1024
_KT = 1024

_NC = 2
_NS = 16
_NW = _NC * _NS
_RPW = _CB // _NW
_LANE = 16
_RUNROLL = 4


def _argmin_block(r_ref, cb_ref, rn_ref, cn_ref, idx_ref):
    r2 = r_ref[...] * 2.0
    rn = rn_ref[...]
    runmin = jnp.full((_BB,), jnp.inf, dtype=jnp.float32)
    runidx = jnp.zeros((_BB,), dtype=jnp.float32)
    colsf = lax.broadcasted_iota(jnp.int32, (_BB, _KT), 1).astype(jnp.float32)
    for kt in range(_K // _KT):
        cb_t = cb_ref[kt * _KT:(kt + 1) * _KT, :]
        m2 = lax.dot_general(r2, cb_t, (((1,), (1,)), ((), ())),
                             preferred_element_type=jnp.float32)
        d = (rn - m2) + cn_ref[:, kt * _KT:(kt + 1) * _KT]
        tmin = jnp.min(d, axis=1)
        tidx = jnp.min(jnp.where(d == tmin[:, None], colsf, jnp.float32(_K)),
                       axis=1) + jnp.float32(kt * _KT)
        upd = tmin < runmin
        runmin = jnp.where(upd, tmin, runmin)
        runidx = jnp.where(upd, tidx, runidx)
    idx_ref[0, 0, :] = runidx.astype(jnp.int32)


def _argmin_call(residual, cb, rnorm, cnorm):
    idx3 = pl.pallas_call(
        _argmin_block,
        grid=(_CB // _BB,),
        in_specs=[
            pl.BlockSpec((_BB, _D), lambda b: (b, 0)),
            pl.BlockSpec((_K, _D), lambda b: (0, 0)),
            pl.BlockSpec((_BB, 1), lambda b: (b, 0)),
            pl.BlockSpec((1, _K), lambda b: (0, 0)),
        ],
        out_specs=pl.BlockSpec((1, 1, _BB), lambda b: (b, 0, 0)),
        out_shape=jax.ShapeDtypeStruct((_CB // _BB, 1, _BB), jnp.int32),
    )(residual, cb, rnorm, cnorm)
    return idx3.reshape(_CB)


def _sc_body(cb_hbm, idx_hbm, r_hbm, qs_hbm, rn_hbm, cp_hbm,
             idx_v, q_v, r_v, qs_v, acc_v, sem):
    wid = lax.axis_index("s") * _NC + lax.axis_index("c")
    base = wid * _RPW
    pltpu.sync_copy(idx_hbm.at[pl.ds(base, _RPW)], idx_v)
    pltpu.sync_copy(r_hbm.at[pl.ds(base, _RPW)], r_v)
    pltpu.async_copy(cb_hbm.at[idx_v], q_v, sem).wait()
    acc_v[...] = jnp.zeros((_LANE,), jnp.float32)

    def row_body(i, _):
        for u in range(_RUNROLL):
            row = i * _RUNROLL + u
            for c in range(_D // _LANE):
                sl = pl.ds(c * _LANE, _LANE)
                q = q_v[row, sl]
                r = r_v[row, sl]
                t = q - r
                acc_v[...] = acc_v[...] + t * t
                qs = r + t
                qs_v[row, sl] = qs
                r_v[row, sl] = r - qs
        return _

    lax.fori_loop(0, _RPW // _RUNROLL, row_body, None)
    pltpu.sync_copy(qs_v, qs_hbm.at[pl.ds(base, _RPW)])
    pltpu.sync_copy(r_v, rn_hbm.at[pl.ds(base, _RPW)])
    pltpu.sync_copy(acc_v, cp_hbm.at[wid])


_sc_update = pl.kernel(
    _sc_body,
    out_type=(
        jax.ShapeDtypeStruct((_CB, _D), jnp.float32),
        jax.ShapeDtypeStruct((_CB, _D), jnp.float32),
        jax.ShapeDtypeStruct((_NW, _LANE), jnp.float32),
    ),
    mesh=plsc.VectorSubcoreMesh(core_axis_name="c", subcore_axis_name="s",
                                num_cores=_NC, num_subcores=_NS),
    scratch_types=(
        pltpu.VMEM((_RPW,), jnp.int32),
        pltpu.VMEM((_RPW, 2 * _D), jnp.float32),
        pltpu.VMEM((_RPW, _D), jnp.float32),
        pltpu.VMEM((_RPW, _D), jnp.float32),
        pltpu.VMEM((_LANE,), jnp.float32),
        pltpu.SemaphoreType.DMA,
    ),
)


def kernel(x, cb0, cb1, cb2):
    cbs = (cb0, cb1, cb2)
    cnorms = [jnp.sum(cb ** 2, axis=1)[None, :] for cb in cbs]
    cb_pads = [jnp.pad(cb, ((0, 0), (0, _D))) for cb in cbs]

    residuals = [x[c * _CB:(c + 1) * _CB] for c in range(_NCHUNK)]
    q_stes = [[] for _ in range(_NCHUNK)]
    indices = [[] for _ in range(_NCHUNK)]
    commit_sums = []
    for lvl in range(3):
        cps = []
        for c in range(_NCHUNK):
            r = residuals[c]
            rnorm = jnp.sum(r ** 2, axis=1, keepdims=True)
            idx = _argmin_call(r, cbs[lvl], rnorm, cnorms[lvl])
            q_ste, r_new, cp = _sc_update(cb_pads[lvl], idx, r)
            residuals[c] = r_new
            q_stes[c].append(q_ste)
            indices[c].append(idx)
            cps.append(cp)
        commit_sums.append(sum(jnp.sum(cp) for cp in cps))

    quantized_sum = jnp.concatenate(
        [(q_stes[c][0] + q_stes[c][1]) + q_stes[c][2] for c in range(_NCHUNK)])
    idx_stacked = jnp.stack(
        [jnp.concatenate([indices[c][lvl] for c in range(_NCHUNK)])
         for lvl in range(3)], axis=0)
    total_commitment_loss = jnp.float32(0.0)
    for s in commit_sums:
        total_commitment_loss = total_commitment_loss + s / jnp.float32(_B * _D)
    reconstruction_loss = jnp.mean((quantized_sum - x) ** 2)
    total_loss = reconstruction_loss + _COMMITMENT_COST * total_commitment_loss
    return (quantized_sum, idx_stacked,
            reconstruction_loss, total_commitment_loss, total_loss)


# speedup vs baseline: 1.0887x; 1.0887x over previous
import jax
import jax.numpy as jnp
from jax import lax
from jax.experimental import pallas as pl
from jax.experimental.pallas import tpu as pltpu
from jax.experimental.pallas import tpu_sc as plsc

_B = 4096
_D = 64
_K = 8192
_COMMITMENT_COST = 0.25

_NCHUNK = 1
_CB = _B // _NCHUNK

_BB = 1024
_KT = 1024

_NC = 2
_NS = 16
_NW = _NC * _NS
_RPW = _CB // _NW
_LANE = 16
_RUNROLL = 4


def _argmin_block(r_ref, cb_ref, cn_ref, idx_ref):
    r = r_ref[...]
    r2 = r * 2.0
    rn = jnp.sum(r * r, axis=1)[:, None]
    runmin = jnp.full((_BB,), jnp.inf, dtype=jnp.float32)
    runidx = jnp.zeros((_BB,), dtype=jnp.float32)
    colsf = lax.broadcasted_iota(jnp.int32, (_BB, _KT), 1).astype(jnp.float32)
    for kt in range(_K // _KT):
        cb_t = cb_ref[kt * _KT:(kt + 1) * _KT, :]
        m2 = lax.dot_general(r2, cb_t, (((1,), (1,)), ((), ())),
                             preferred_element_type=jnp.float32)
        d = (rn - m2) + cn_ref[:, kt * _KT:(kt + 1) * _KT]
        tmin = jnp.min(d, axis=1)
        tidx = jnp.min(jnp.where(d == tmin[:, None], colsf, jnp.float32(_K)),
                       axis=1) + jnp.float32(kt * _KT)
        upd = tmin < runmin
        runmin = jnp.where(upd, tmin, runmin)
        runidx = jnp.where(upd, tidx, runidx)
    idx_ref[0, 0, :] = runidx.astype(jnp.int32)


def _argmin_call(residual, cb, cnorm):
    idx3 = pl.pallas_call(
        _argmin_block,
        grid=(_CB // _BB,),
        in_specs=[
            pl.BlockSpec((_BB, _D), lambda b: (b, 0)),
            pl.BlockSpec((_K, _D), lambda b: (0, 0)),
            pl.BlockSpec((1, _K), lambda b: (0, 0)),
        ],
        out_specs=pl.BlockSpec((1, 1, _BB), lambda b: (b, 0, 0)),
        out_shape=jax.ShapeDtypeStruct((_CB // _BB, 1, _BB), jnp.int32),
    )(residual, cb, cnorm)
    return idx3.reshape(_CB)


def _sc_body(cb_hbm, idx_hbm, r_hbm, qs_hbm, rn_hbm, cp_hbm,
             idx_v, q_v, r_v, qs_v, acc_v, sem):
    wid = lax.axis_index("s") * _NC + lax.axis_index("c")
    base = wid * _RPW
    pltpu.sync_copy(idx_hbm.at[pl.ds(base, _RPW)], idx_v)
    pltpu.sync_copy(r_hbm.at[pl.ds(base, _RPW)], r_v)
    pltpu.async_copy(cb_hbm.at[idx_v], q_v, sem).wait()
    acc_v[...] = jnp.zeros((_LANE,), jnp.float32)

    def row_body(i, _):
        for u in range(_RUNROLL):
            row = i * _RUNROLL + u
            for c in range(_D // _LANE):
                sl = pl.ds(c * _LANE, _LANE)
                q = q_v[row, sl]
                r = r_v[row, sl]
                t = q - r
                acc_v[...] = acc_v[...] + t * t
                qs = r + t
                qs_v[row, sl] = qs
                r_v[row, sl] = r - qs
        return _

    lax.fori_loop(0, _RPW // _RUNROLL, row_body, None)
    pltpu.sync_copy(qs_v, qs_hbm.at[pl.ds(base, _RPW)])
    pltpu.sync_copy(r_v, rn_hbm.at[pl.ds(base, _RPW)])
    pltpu.sync_copy(acc_v, cp_hbm.at[wid])


_sc_update = pl.kernel(
    _sc_body,
    out_type=(
        jax.ShapeDtypeStruct((_CB, _D), jnp.float32),
        jax.ShapeDtypeStruct((_CB, _D), jnp.float32),
        jax.ShapeDtypeStruct((_NW, _LANE), jnp.float32),
    ),
    mesh=plsc.VectorSubcoreMesh(core_axis_name="c", subcore_axis_name="s",
                                num_cores=_NC, num_subcores=_NS),
    scratch_types=(
        pltpu.VMEM((_RPW,), jnp.int32),
        pltpu.VMEM((_RPW, 2 * _D), jnp.float32),
        pltpu.VMEM((_RPW, _D), jnp.float32),
        pltpu.VMEM((_RPW, _D), jnp.float32),
        pltpu.VMEM((_LANE,), jnp.float32),
        pltpu.SemaphoreType.DMA,
    ),
)


def kernel(x, cb0, cb1, cb2):
    cbs = (cb0, cb1, cb2)
    cnorms = [jnp.sum(cb ** 2, axis=1)[None, :] for cb in cbs]
    cb_pads = [jnp.pad(cb, ((0, 0), (0, _D))) for cb in cbs]

    residuals = [x[c * _CB:(c + 1) * _CB] for c in range(_NCHUNK)]
    q_stes = [[] for _ in range(_NCHUNK)]
    indices = [[] for _ in range(_NCHUNK)]
    commit_sums = []
    for lvl in range(3):
        cps = []
        for c in range(_NCHUNK):
            r = residuals[c]
            idx = _argmin_call(r, cbs[lvl], cnorms[lvl])
            q_ste, r_new, cp = _sc_update(cb_pads[lvl], idx, r)
            residuals[c] = r_new
            q_stes[c].append(q_ste)
            indices[c].append(idx)
            cps.append(cp)
        commit_sums.append(sum(jnp.sum(cp) for cp in cps))

    quantized_sum = jnp.concatenate(
        [(q_stes[c][0] + q_stes[c][1]) + q_stes[c][2] for c in range(_NCHUNK)])
    idx_stacked = jnp.stack(
        [jnp.concatenate([indices[c][lvl] for c in range(_NCHUNK)])
         for lvl in range(3)], axis=0)
    total_commitment_loss = jnp.float32(0.0)
    for s in commit_sums:
        total_commitment_loss = total_commitment_loss + s / jnp.float32(_B * _D)
    reconstruction_loss = jnp.mean((quantized_sum - x) ** 2)
    total_loss = reconstruction_loss + _COMMITMENT_COST * total_commitment_loss
    return (quantized_sum, idx_stacked,
            reconstruction_loss, total_commitment_loss, total_loss)


# speedup vs baseline: 1.1504x; 1.0567x over previous
import jax
import jax.numpy as jnp
from jax import lax
from jax.experimental import pallas as pl
from jax.experimental.pallas import tpu as pltpu
from jax.experimental.pallas import tpu_sc as plsc

_B = 4096
_D = 64
_K = 8192
_COMMITMENT_COST = 0.25

_NCHUNK = 1
_CB = _B // _NCHUNK

_BB = 1024
_KT = 1024

_NC = 2
_NS = 16
_NW = _NC * _NS
_RPW = _CB // _NW
_LANE = 16
_RUNROLL = 4


def _argmin_block(r_ref, cb_ref, cn_ref, idx_ref):
    r = r_ref[...]
    r2 = r * 2.0
    rn = jnp.sum(r * r, axis=1)[:, None]
    runmin = jnp.full((_BB,), jnp.inf, dtype=jnp.float32)
    runidx = jnp.zeros((_BB,), dtype=jnp.float32)
    colsf = lax.broadcasted_iota(jnp.int32, (_BB, _KT), 1).astype(jnp.float32)
    for kt in range(_K // _KT):
        cb_t = cb_ref[kt * _KT:(kt + 1) * _KT, :]
        m2 = lax.dot_general(r2, cb_t, (((1,), (1,)), ((), ())),
                             preferred_element_type=jnp.float32)
        d = (rn - m2) + cn_ref[:, kt * _KT:(kt + 1) * _KT]
        tmin = jnp.min(d, axis=1)
        tidx = jnp.min(jnp.where(d == tmin[:, None], colsf, jnp.float32(_K)),
                       axis=1) + jnp.float32(kt * _KT)
        upd = tmin < runmin
        runmin = jnp.where(upd, tmin, runmin)
        runidx = jnp.where(upd, tidx, runidx)
    idx_ref[0, 0, :] = runidx.astype(jnp.int32)


def _argmin_call(residual, cb, cnorm):
    idx3 = pl.pallas_call(
        _argmin_block,
        grid=(_CB // _BB,),
        in_specs=[
            pl.BlockSpec((_BB, _D), lambda b: (b, 0)),
            pl.BlockSpec((_K, _D), lambda b: (0, 0)),
            pl.BlockSpec((1, _K), lambda b: (0, 0)),
        ],
        out_specs=pl.BlockSpec((1, 1, _BB), lambda b: (b, 0, 0)),
        out_shape=jax.ShapeDtypeStruct((_CB // _BB, 1, _BB), jnp.int32),
    )(residual, cb, cnorm)
    return idx3.reshape(_CB)


def _sc_body(cb_hbm, idx_hbm, r_hbm, qs_hbm, rn_hbm, cp_hbm,
             idx_v, q_v, r_v, qs_v, acc_v, sem_g, sem_r):
    wid = lax.axis_index("s") * _NC + lax.axis_index("c")
    base = wid * _RPW
    r_cp = pltpu.async_copy(r_hbm.at[pl.ds(base, _RPW)], r_v, sem_r)
    pltpu.sync_copy(idx_hbm.at[pl.ds(base, _RPW)], idx_v)
    g_cp = pltpu.async_copy(cb_hbm.at[idx_v], q_v, sem_g)
    r_cp.wait()
    g_cp.wait()

    def row_body(i, acc):
        for u in range(_RUNROLL):
            row = i * _RUNROLL + u
            for c in range(_D // _LANE):
                sl = pl.ds(c * _LANE, _LANE)
                q = q_v[row, sl]
                r = r_v[row, sl]
                t = q - r
                acc = acc + t * t
                qs = r + t
                qs_v[row, sl] = qs
                r_v[row, sl] = r - qs
        return acc

    acc = lax.fori_loop(0, _RPW // _RUNROLL, row_body,
                        jnp.zeros((_LANE,), jnp.float32))
    acc_v[...] = acc
    q_out = pltpu.async_copy(qs_v, qs_hbm.at[pl.ds(base, _RPW)], sem_g)
    r_out = pltpu.async_copy(r_v, rn_hbm.at[pl.ds(base, _RPW)], sem_r)
    pltpu.sync_copy(acc_v, cp_hbm.at[wid])
    q_out.wait()
    r_out.wait()


_sc_update = pl.kernel(
    _sc_body,
    out_type=(
        jax.ShapeDtypeStruct((_CB, _D), jnp.float32),
        jax.ShapeDtypeStruct((_CB, _D), jnp.float32),
        jax.ShapeDtypeStruct((_NW, _LANE), jnp.float32),
    ),
    mesh=plsc.VectorSubcoreMesh(core_axis_name="c", subcore_axis_name="s",
                                num_cores=_NC, num_subcores=_NS),
    scratch_types=(
        pltpu.VMEM((_RPW,), jnp.int32),
        pltpu.VMEM((_RPW, 2 * _D), jnp.float32),
        pltpu.VMEM((_RPW, _D), jnp.float32),
        pltpu.VMEM((_RPW, _D), jnp.float32),
        pltpu.VMEM((_LANE,), jnp.float32),
        pltpu.SemaphoreType.DMA,
        pltpu.SemaphoreType.DMA,
    ),
)


def kernel(x, cb0, cb1, cb2):
    cbs = (cb0, cb1, cb2)
    cnorms = [jnp.sum(cb ** 2, axis=1)[None, :] for cb in cbs]
    cb_pads = [jnp.pad(cb, ((0, 0), (0, _D))) for cb in cbs]

    residuals = [x[c * _CB:(c + 1) * _CB] for c in range(_NCHUNK)]
    q_stes = [[] for _ in range(_NCHUNK)]
    indices = [[] for _ in range(_NCHUNK)]
    commit_sums = []
    for lvl in range(3):
        cps = []
        for c in range(_NCHUNK):
            r = residuals[c]
            idx = _argmin_call(r, cbs[lvl], cnorms[lvl])
            q_ste, r_new, cp = _sc_update(cb_pads[lvl], idx, r)
            residuals[c] = r_new
            q_stes[c].append(q_ste)
            indices[c].append(idx)
            cps.append(cp)
        commit_sums.append(sum(jnp.sum(cp) for cp in cps))

    quantized_sum = jnp.concatenate(
        [(q_stes[c][0] + q_stes[c][1]) + q_stes[c][2] for c in range(_NCHUNK)])
    idx_stacked = jnp.stack(
        [jnp.concatenate([indices[c][lvl] for c in range(_NCHUNK)])
         for lvl in range(3)], axis=0)
    total_commitment_loss = jnp.float32(0.0)
    for s in commit_sums:
        total_commitment_loss = total_commitment_loss + s / jnp.float32(_B * _D)
    reconstruction_loss = jnp.mean((quantized_sum - x) ** 2)
    total_loss = reconstruction_loss + _COMMITMENT_COST * total_commitment_loss
    return (quantized_sum, idx_stacked,
            reconstruction_loss, total_commitment_loss, total_loss)


# speedup vs baseline: 1.1950x; 1.0387x over previous
import jax
import jax.numpy as jnp
from jax import lax
from jax.experimental import pallas as pl
from jax.experimental.pallas import tpu as pltpu
from jax.experimental.pallas import tpu_sc as plsc

_B = 4096
_D = 64
_K = 8192
_COMMITMENT_COST = 0.25

_NCHUNK = 1
_CB = _B // _NCHUNK

_BB = 1024
_KT = 1024

_NC = 2
_NS = 16
_NW = _NC * _NS
_RPW = _CB // _NW
_LANE = 16
_RUNROLL = 4


def _argmin_block(r_ref, cb_ref, cn_ref, idx_ref, pad_ref):
    @pl.when(pl.program_id(0) == 0)
    def _write_padded_codebook():
        pad_ref[:, 0:_D] = cb_ref[...]

    r = r_ref[...]
    r2 = r * 2.0
    rn = jnp.sum(r * r, axis=1)[:, None]
    runmin = jnp.full((_BB,), jnp.inf, dtype=jnp.float32)
    runidx = jnp.zeros((_BB,), dtype=jnp.float32)
    colsf = lax.broadcasted_iota(jnp.int32, (_BB, _KT), 1).astype(jnp.float32)
    for kt in range(_K // _KT):
        cb_t = cb_ref[kt * _KT:(kt + 1) * _KT, :]
        m2 = lax.dot_general(r2, cb_t, (((1,), (1,)), ((), ())),
                             preferred_element_type=jnp.float32)
        d = (rn - m2) + cn_ref[:, kt * _KT:(kt + 1) * _KT]
        tmin = jnp.min(d, axis=1)
        tidx = jnp.min(jnp.where(d == tmin[:, None], colsf, jnp.float32(_K)),
                       axis=1) + jnp.float32(kt * _KT)
        upd = tmin < runmin
        runmin = jnp.where(upd, tmin, runmin)
        runidx = jnp.where(upd, tidx, runidx)
    idx_ref[0, 0, :] = runidx.astype(jnp.int32)


def _argmin_call(residual, cb, cnorm):
    idx3 = pl.pallas_call(
        _argmin_block,
        grid=(_CB // _BB,),
        in_specs=[
            pl.BlockSpec((_BB, _D), lambda b: (b, 0)),
            pl.BlockSpec((_K, _D), lambda b: (0, 0)),
            pl.BlockSpec((1, _K), lambda b: (0, 0)),
        ],
        out_specs=[
            pl.BlockSpec((1, 1, _BB), lambda b: (b, 0, 0)),
            pl.BlockSpec((_K, 2 * _D), lambda b: (0, 0)),
        ],
        out_shape=[
            jax.ShapeDtypeStruct((_CB // _BB, 1, _BB), jnp.int32),
            jax.ShapeDtypeStruct((_K, 2 * _D), jnp.float32),
        ],
    )(residual, cb, cnorm)
    idx3, cb_pad = idx3
    return idx3.reshape(_CB), cb_pad


def _sc_body(cb_hbm, idx_hbm, r_hbm, qs_hbm, rn_hbm, cp_hbm,
             idx_v, q_v, r_v, qs_v, acc_v, sem_g, sem_r):
    wid = lax.axis_index("s") * _NC + lax.axis_index("c")
    base = wid * _RPW
    r_cp = pltpu.async_copy(r_hbm.at[pl.ds(base, _RPW)], r_v, sem_r)
    pltpu.sync_copy(idx_hbm.at[pl.ds(base, _RPW)], idx_v)
    g_cp = pltpu.async_copy(cb_hbm.at[idx_v], q_v, sem_g)
    r_cp.wait()
    g_cp.wait()

    def row_body(i, acc):
        for u in range(_RUNROLL):
            row = i * _RUNROLL + u
            for c in range(_D // _LANE):
                sl = pl.ds(c * _LANE, _LANE)
                q = q_v[row, sl]
                r = r_v[row, sl]
                t = q - r
                acc = acc + t * t
                qs = r + t
                qs_v[row, sl] = qs
                r_v[row, sl] = r - qs
        return acc

    acc = lax.fori_loop(0, _RPW // _RUNROLL, row_body,
                        jnp.zeros((_LANE,), jnp.float32))
    acc_v[...] = acc
    q_out = pltpu.async_copy(qs_v, qs_hbm.at[pl.ds(base, _RPW)], sem_g)
    r_out = pltpu.async_copy(r_v, rn_hbm.at[pl.ds(base, _RPW)], sem_r)
    pltpu.sync_copy(acc_v, cp_hbm.at[wid])
    q_out.wait()
    r_out.wait()


_sc_update = pl.kernel(
    _sc_body,
    out_type=(
        jax.ShapeDtypeStruct((_CB, _D), jnp.float32),
        jax.ShapeDtypeStruct((_CB, _D), jnp.float32),
        jax.ShapeDtypeStruct((_NW, _LANE), jnp.float32),
    ),
    mesh=plsc.VectorSubcoreMesh(core_axis_name="c", subcore_axis_name="s",
                                num_cores=_NC, num_subcores=_NS),
    scratch_types=(
        pltpu.VMEM((_RPW,), jnp.int32),
        pltpu.VMEM((_RPW, 2 * _D), jnp.float32),
        pltpu.VMEM((_RPW, _D), jnp.float32),
        pltpu.VMEM((_RPW, _D), jnp.float32),
        pltpu.VMEM((_LANE,), jnp.float32),
        pltpu.SemaphoreType.DMA,
        pltpu.SemaphoreType.DMA,
    ),
)


def kernel(x, cb0, cb1, cb2):
    cbs = (cb0, cb1, cb2)
    cnorms = [jnp.sum(cb ** 2, axis=1)[None, :] for cb in cbs]

    residuals = [x[c * _CB:(c + 1) * _CB] for c in range(_NCHUNK)]
    q_stes = [[] for _ in range(_NCHUNK)]
    indices = [[] for _ in range(_NCHUNK)]
    commit_sums = []
    for lvl in range(3):
        cps = []
        for c in range(_NCHUNK):
            r = residuals[c]
            idx, cb_pad = _argmin_call(r, cbs[lvl], cnorms[lvl])
            q_ste, r_new, cp = _sc_update(cb_pad, idx, r)
            residuals[c] = r_new
            q_stes[c].append(q_ste)
            indices[c].append(idx)
            cps.append(cp)
        commit_sums.append(sum(jnp.sum(cp) for cp in cps))

    quantized_sum = jnp.concatenate(
        [(q_stes[c][0] + q_stes[c][1]) + q_stes[c][2] for c in range(_NCHUNK)])
    idx_stacked = jnp.stack(
        [jnp.concatenate([indices[c][lvl] for c in range(_NCHUNK)])
         for lvl in range(3)], axis=0)
    total_commitment_loss = jnp.float32(0.0)
    for s in commit_sums:
        total_commitment_loss = total_commitment_loss + s / jnp.float32(_B * _D)
    reconstruction_loss = jnp.mean((quantized_sum - x) ** 2)
    total_loss = reconstruction_loss + _COMMITMENT_COST * total_commitment_loss
    return (quantized_sum, idx_stacked,
            reconstruction_loss, total_commitment_loss, total_loss)
